# initial kernel scaffold (unmeasured)
import jax
import jax.numpy as jnp
from jax import lax
from jax.experimental import pallas as pl
from jax.experimental.pallas import tpu as pltpu


def kernel(
    x,
):
    def body(*refs):
        pass

    out_shape = jax.ShapeDtypeStruct(..., jnp.float32)
    return pl.pallas_call(body, out_shape=out_shape)(...)


# baseline (device time: 17482 ns/iter reference)
import jax
import jax.numpy as jnp
from jax import lax
from jax.experimental import pallas as pl
from jax.experimental.pallas import tpu as pltpu

N_DEV = 16


def kernel(x):
    m, n = x.shape

    def body(x_ref, out_ref, comm_ref, send_sems, recv_sems):
        my = lax.axis_index("i")

        partial = jnp.sum(x_ref[...], axis=0, keepdims=True)
        comm_ref[pl.ds(my, 1), :] = partial

        for t in range(N_DEV):
            @pl.when(t != my)
            def _():
                rdma = pltpu.make_async_remote_copy(
                    src_ref=comm_ref.at[pl.ds(my, 1)],
                    dst_ref=comm_ref.at[pl.ds(my, 1)],
                    send_sem=send_sems.at[t],
                    recv_sem=recv_sems.at[my],
                    device_id=(t,),
                    device_id_type=pl.DeviceIdType.MESH,
                )
                rdma.start()

        for j in range(N_DEV):
            @pl.when(j != my)
            def _():
                recv = pltpu.make_async_remote_copy(
                    src_ref=comm_ref.at[pl.ds(j, 1)],
                    dst_ref=comm_ref.at[pl.ds(j, 1)],
                    send_sem=send_sems.at[j],
                    recv_sem=recv_sems.at[j],
                    device_id=(j,),
                    device_id_type=pl.DeviceIdType.MESH,
                )
                recv.wait_recv()
                send = pltpu.make_async_remote_copy(
                    src_ref=comm_ref.at[pl.ds(my, 1)],
                    dst_ref=comm_ref.at[pl.ds(my, 1)],
                    send_sem=send_sems.at[j],
                    recv_sem=recv_sems.at[j],
                    device_id=(j,),
                    device_id_type=pl.DeviceIdType.MESH,
                )
                send.wait_send()

        out_ref[...] = jnp.sum(comm_ref[...], axis=0, keepdims=True)

    return pl.pallas_call(
        body,
        out_shape=jax.ShapeDtypeStruct((1, n), jnp.float32),
        in_specs=[pl.BlockSpec(memory_space=pltpu.VMEM)],
        out_specs=pl.BlockSpec(memory_space=pltpu.VMEM),
        scratch_shapes=[
            pltpu.VMEM((N_DEV, n), jnp.float32),
            pltpu.SemaphoreType.DMA((N_DEV,)),
            pltpu.SemaphoreType.DMA((N_DEV,)),
        ],
    )(x)


# device time: 11775 ns/iter; 1.4847x vs baseline; 1.4847x over previous
import jax
import jax.numpy as jnp
from jax import lax
from jax.experimental import pallas as pl
from jax.experimental.pallas import tpu as pltpu

N_DEV = 16
N_CHUNKS = 4


def kernel(x):
    m, n = x.shape
    ch = m // N_CHUNKS

    def body(x_ref, out_ref, buf_ref, copy_sems, comm_ref, send_sems, recv_sems):
        my = lax.axis_index("i")

        barrier_sem = pltpu.get_barrier_semaphore()
        for t in range(N_DEV):
            @pl.when(t != my)
            def _():
                pl.semaphore_signal(
                    barrier_sem, inc=1,
                    device_id=(t,), device_id_type=pl.DeviceIdType.MESH,
                )

        def copy_chunk(c):
            return pltpu.make_async_copy(
                x_ref.at[pl.ds(c * ch, ch), :],
                buf_ref.at[c % 2],
                copy_sems.at[c % 2],
            )

        copy_chunk(0).start()
        acc = None
        for c in range(N_CHUNKS):
            copy_chunk(c).wait()
            if c + 1 < N_CHUNKS:
                copy_chunk(c + 1).start()
            s = jnp.sum(buf_ref[c % 2], axis=0, keepdims=True)
            acc = s if acc is None else acc + s
        comm_ref[pl.ds(my, 1), :] = acc

        pl.semaphore_wait(barrier_sem, N_DEV - 1)

        for t in range(N_DEV):
            @pl.when(t != my)
            def _():
                rdma = pltpu.make_async_remote_copy(
                    src_ref=comm_ref.at[pl.ds(my, 1)],
                    dst_ref=comm_ref.at[pl.ds(my, 1)],
                    send_sem=send_sems.at[t],
                    recv_sem=recv_sems.at[my],
                    device_id=(t,),
                    device_id_type=pl.DeviceIdType.MESH,
                )
                rdma.start()

        for j in range(N_DEV):
            @pl.when(j != my)
            def _():
                recv = pltpu.make_async_remote_copy(
                    src_ref=comm_ref.at[pl.ds(j, 1)],
                    dst_ref=comm_ref.at[pl.ds(j, 1)],
                    send_sem=send_sems.at[j],
                    recv_sem=recv_sems.at[j],
                    device_id=(j,),
                    device_id_type=pl.DeviceIdType.MESH,
                )
                recv.wait_recv()
                send = pltpu.make_async_remote_copy(
                    src_ref=comm_ref.at[pl.ds(my, 1)],
                    dst_ref=comm_ref.at[pl.ds(my, 1)],
                    send_sem=send_sems.at[j],
                    recv_sem=recv_sems.at[j],
                    device_id=(j,),
                    device_id_type=pl.DeviceIdType.MESH,
                )
                send.wait_send()

        out_ref[...] = jnp.sum(comm_ref[...], axis=0, keepdims=True)

    return pl.pallas_call(
        body,
        out_shape=jax.ShapeDtypeStruct((1, n), jnp.float32),
        in_specs=[pl.BlockSpec(memory_space=pl.ANY)],
        out_specs=pl.BlockSpec(memory_space=pltpu.VMEM),
        scratch_shapes=[
            pltpu.VMEM((2, ch, n), jnp.float32),
            pltpu.SemaphoreType.DMA((2,)),
            pltpu.VMEM((N_DEV, n), jnp.float32),
            pltpu.SemaphoreType.DMA((N_DEV,)),
            pltpu.SemaphoreType.DMA((N_DEV,)),
        ],
        compiler_params=pltpu.CompilerParams(collective_id=0),
    )(x)
